# baseline (speedup 1.0000x reference)
import functools

import jax
import jax.numpy as jnp
from jax import lax
from jax.experimental import pallas as pl
from jax.experimental.pallas import tpu as pltpu
from jax.experimental.pallas import tpu_sc as plsc

NUM_USERS = 30000
NUM_ITEMS = 20000
D = 64
SBERT = 384
NUM_LAYERS = 3
E = 800000
N = NUM_USERS + NUM_ITEMS
NPAD = 51200
HALF = 25000
TRASH = 25000
ACC_ROWS = 25024
ROWS = 6400
EPAD = ROWS * 128
NB = 8
NBLK = NPAD // NB

_mesh = plsc.VectorSubcoreMesh(core_axis_name="c", subcore_axis_name="s")


def _proj_body(x_ref, w_ref, o_ref):
    o_ref[...] = jnp.dot(x_ref[...], w_ref[...],
                         preferred_element_type=jnp.float32,
                         precision=lax.Precision.HIGHEST)


def _book_proj(book_bert, w_t):
    return pl.pallas_call(
        _proj_body,
        grid=(10,),
        in_specs=[pl.BlockSpec((2000, SBERT), lambda i: (i, 0)),
                  pl.BlockSpec((SBERT, D), lambda i: (0, 0))],
        out_specs=pl.BlockSpec((2000, D), lambda i: (i, 0)),
        out_shape=jax.ShapeDtypeStruct((NUM_ITEMS, D), jnp.float32),
    )(book_bert, w_t)


def _dinv_body(degp_ref, emb_ref, dinv_ref, z_ref):
    deg = jnp.sum(degp_ref[...], axis=1, keepdims=True)
    dinv = jnp.where(deg > 0, lax.rsqrt(deg), 0.0)
    dinv_ref[...] = dinv
    z_ref[...] = emb_ref[...] * dinv


def _dinv_z0(degp, emb0p):
    return pl.pallas_call(
        _dinv_body,
        grid=(NB,),
        in_specs=[pl.BlockSpec((NBLK, 32), lambda i: (i, 0)),
                  pl.BlockSpec((NBLK, D), lambda i: (i, 0))],
        out_specs=[pl.BlockSpec((NBLK, 1), lambda i: (i, 0)),
                   pl.BlockSpec((NBLK, D), lambda i: (i, 0))],
        out_shape=[jax.ShapeDtypeStruct((NPAD, 1), jnp.float32),
                   jax.ShapeDtypeStruct((NPAD, D), jnp.float32)],
    )(degp, emb0p)


def _scale_body(y_ref, dinv_ref, s_ref, so_ref, z_ref):
    x = y_ref[...] * dinv_ref[...]
    so_ref[...] = s_ref[...] + x
    z_ref[...] = x * dinv_ref[...]


def _scale_accum(y, dinv, s):
    return pl.pallas_call(
        _scale_body,
        grid=(NB,),
        in_specs=[pl.BlockSpec((NBLK, D), lambda i: (i, 0)),
                  pl.BlockSpec((NBLK, 1), lambda i: (i, 0)),
                  pl.BlockSpec((NBLK, D), lambda i: (i, 0))],
        out_specs=[pl.BlockSpec((NBLK, D), lambda i: (i, 0)),
                   pl.BlockSpec((NBLK, D), lambda i: (i, 0))],
        out_shape=[jax.ShapeDtypeStruct((NPAD, D), jnp.float32),
                   jax.ShapeDtypeStruct((NPAD, D), jnp.float32)],
    )(y, dinv, s)


def _final_body(y_ref, dinv_ref, s_ref, o_ref):
    o_ref[...] = (s_ref[...] + y_ref[...] * dinv_ref[...]) * 0.25


def _final_mean(y, dinv, s):
    return pl.pallas_call(
        _final_body,
        grid=(NB,),
        in_specs=[pl.BlockSpec((NBLK, D), lambda i: (i, 0)),
                  pl.BlockSpec((NBLK, 1), lambda i: (i, 0)),
                  pl.BlockSpec((NBLK, D), lambda i: (i, 0))],
        out_specs=pl.BlockSpec((NBLK, D), lambda i: (i, 0)),
        out_shape=jax.ShapeDtypeStruct((NPAD, D), jnp.float32),
    )(y, dinv, s)


@functools.partial(
    pl.kernel,
    out_type=(jax.ShapeDtypeStruct((32, NPAD // 128, 128), jnp.float32),
              jax.ShapeDtypeStruct((2, ROWS, 128), jnp.int32)),
    mesh=_mesh,
    scratch_types=[pltpu.VMEM((NPAD // 128, 128), jnp.float32),
                   pltpu.VMEM((8, 128), jnp.int32),
                   pltpu.VMEM((8, 128), jnp.int32),
                   pltpu.VMEM((8, 128), jnp.int32)],
    compiler_params=pltpu.CompilerParams(needs_layout_passes=False,
                                         use_tc_tiling_on_sc=False),
)
def _deg_kernel(dst_hbm, degp_hbm, dstloc_hbm, degpart, dstbuf, dl0buf, dl1buf):
    cid = lax.axis_index("c")
    sid = lax.axis_index("s")
    w = cid * 16 + sid

    z16 = jnp.zeros((16,), jnp.float32)

    def zero_body(i, _):
        for k in range(8):
            degpart[i, pl.ds(k * 16, 16)] = z16
        return 0

    lax.fori_loop(0, NPAD // 128, zero_body, 0)

    ones = jnp.ones((16,), jnp.float32)
    halfv = jnp.full((16,), HALF, jnp.int32)
    trashv = jnp.full((16,), TRASH, jnp.int32)

    def chunk_body(n, _):
        row0 = w * 200 + n * 8
        pltpu.sync_copy(dst_hbm.at[pl.ds(row0, 8)], dstbuf)
        for r in range(8):
            for k in range(8):
                d = dstbuf[r, pl.ds(k * 16, 16)]
                plsc.addupdate_scatter(degpart, [d >> 7, d & 127], ones)
                dl0buf[r, pl.ds(k * 16, 16)] = jnp.where(d < halfv, d, trashv)
                dl1buf[r, pl.ds(k * 16, 16)] = jnp.where(d >= halfv, d - halfv,
                                                         trashv)
        pltpu.sync_copy(dl0buf, dstloc_hbm.at[0, pl.ds(row0, 8)])
        pltpu.sync_copy(dl1buf, dstloc_hbm.at[1, pl.ds(row0, 8)])
        return 0

    lax.fori_loop(0, 25, chunk_body, 0)
    pltpu.sync_copy(degpart, degp_hbm.at[w])


@functools.partial(
    pl.kernel,
    out_type=jax.ShapeDtypeStruct((NPAD, D), jnp.float32),
    mesh=_mesh,
    scratch_types=[pltpu.VMEM_SHARED((ACC_ROWS, D), jnp.float32),
                   pltpu.VMEM((8, 128), jnp.int32),
                   pltpu.VMEM((8, 128), jnp.int32),
                   pltpu.VMEM((2, 128, D), jnp.float32),
                   pltpu.SemaphoreType.DMA,
                   pltpu.SemaphoreType.DMA],
    compiler_params=pltpu.CompilerParams(needs_layout_passes=False,
                                         use_tc_tiling_on_sc=False),
)
def _layer_kernel(zeros_hbm, z_hbm, src_hbm, dl_hbm, y_hbm,
                  acc, srcbuf, dlbuf, rowbuf, gsem, ssem):
    cid = lax.axis_index("c")
    sid = lax.axis_index("s")

    pltpu.sync_copy(zeros_hbm, acc.at[pl.ds(sid * 1564, 1564)])
    plsc.subcore_barrier()

    def chunk_body(n, _):
        row0 = sid * 400 + n * 8
        pltpu.sync_copy(src_hbm.at[pl.ds(row0, 8)], srcbuf)
        pltpu.sync_copy(dl_hbm.at[cid, pl.ds(row0, 8)], dlbuf)
        g = [None] * 8
        s = [None] * 8
        g[0] = pltpu.async_copy(z_hbm.at[srcbuf.at[0]], rowbuf.at[0], gsem)
        for r in range(8):
            b = r % 2
            g[r].wait()
            s[r] = pltpu.async_copy(rowbuf.at[b], acc.at[dlbuf.at[r]], ssem,
                                    add=True)
            if r < 7:
                if r >= 1:
                    s[r - 1].wait()
                g[r + 1] = pltpu.async_copy(z_hbm.at[srcbuf.at[r + 1]],
                                            rowbuf.at[1 - b], gsem)
        s[6].wait()
        s[7].wait()
        return 0

    lax.fori_loop(0, 50, chunk_body, 0)
    plsc.subcore_barrier()

    lo = jnp.minimum(sid * 1568, HALF - 1568)
    pltpu.sync_copy(acc.at[pl.ds(lo, 1568)],
                    y_hbm.at[pl.ds(cid * HALF + lo, 1568)])


def kernel(edge_index, user_emb, book_bert, W):
    src = edge_index[0].astype(jnp.int32)
    dst = edge_index[1].astype(jnp.int32)
    npadE = EPAD - E
    src2d = jnp.concatenate(
        [src, jnp.zeros((npadE,), jnp.int32)]).reshape(ROWS, 128)
    dst2d = jnp.concatenate(
        [dst, jnp.full((npadE,), N, jnp.int32)]).reshape(ROWS, 128)

    book_emb = _book_proj(book_bert, W.T)
    emb0 = jnp.concatenate([user_emb, book_emb], axis=0)
    emb0p = jnp.pad(emb0, ((0, NPAD - N), (0, 0)))

    degp, dstloc = _deg_kernel(dst2d)
    degp_t = jnp.reshape(degp, (32, NPAD)).T
    dinv, z = _dinv_z0(degp_t, emb0p)

    s = emb0p
    zeros = jnp.zeros((1564, D), jnp.float32)
    for layer in range(NUM_LAYERS):
        y = _layer_kernel(zeros, z, src2d, dstloc)
        if layer < NUM_LAYERS - 1:
            s, z = _scale_accum(y, dinv, s)
        else:
            out = _final_mean(y, dinv, s)
    return (emb0, out[:N])


# speedup vs baseline: 1.3120x; 1.3120x over previous
import functools

import jax
import jax.numpy as jnp
from jax import lax
from jax.experimental import pallas as pl
from jax.experimental.pallas import tpu as pltpu
from jax.experimental.pallas import tpu_sc as plsc

NUM_USERS = 30000
NUM_ITEMS = 20000
D = 64
SBERT = 384
NUM_LAYERS = 3
E = 800000
N = NUM_USERS + NUM_ITEMS
NPAD = 51200
HALF = 25000
TRASH = 25000
ACC_ROWS = 25024
ROWS = 6400
EPAD = ROWS * 128
NB = 8
NBLK = NPAD // NB

_mesh = plsc.VectorSubcoreMesh(core_axis_name="c", subcore_axis_name="s",
                               num_cores=2, num_subcores=16)


def _proj_body(x_ref, w_ref, o_ref):
    o_ref[...] = jnp.dot(x_ref[...], w_ref[...],
                         preferred_element_type=jnp.float32,
                         precision=lax.Precision.HIGHEST)


def _book_proj(book_bert, w_t):
    return pl.pallas_call(
        _proj_body,
        grid=(10,),
        in_specs=[pl.BlockSpec((2000, SBERT), lambda i: (i, 0)),
                  pl.BlockSpec((SBERT, D), lambda i: (0, 0))],
        out_specs=pl.BlockSpec((2000, D), lambda i: (i, 0)),
        out_shape=jax.ShapeDtypeStruct((NUM_ITEMS, D), jnp.float32),
    )(book_bert, w_t)


def _dinv_body(degp_ref, emb_ref, dinv_ref, z_ref):
    deg = jnp.sum(degp_ref[...], axis=1, keepdims=True)
    dinv = jnp.where(deg > 0, lax.rsqrt(deg), 0.0)
    dinv_ref[...] = dinv
    z_ref[...] = emb_ref[...] * dinv


def _dinv_z0(degp, emb0p):
    return pl.pallas_call(
        _dinv_body,
        grid=(NB,),
        in_specs=[pl.BlockSpec((NBLK, 32), lambda i: (i, 0)),
                  pl.BlockSpec((NBLK, D), lambda i: (i, 0))],
        out_specs=[pl.BlockSpec((NBLK, 1), lambda i: (i, 0)),
                   pl.BlockSpec((NBLK, D), lambda i: (i, 0))],
        out_shape=[jax.ShapeDtypeStruct((NPAD, 1), jnp.float32),
                   jax.ShapeDtypeStruct((NPAD, D), jnp.float32)],
    )(degp, emb0p)


def _scale_body(y_ref, dinv_ref, s_ref, so_ref, z_ref):
    x = y_ref[...] * dinv_ref[...]
    so_ref[...] = s_ref[...] + x
    z_ref[...] = x * dinv_ref[...]


def _scale_accum(y, dinv, s):
    return pl.pallas_call(
        _scale_body,
        grid=(NB,),
        in_specs=[pl.BlockSpec((NBLK, D), lambda i: (i, 0)),
                  pl.BlockSpec((NBLK, 1), lambda i: (i, 0)),
                  pl.BlockSpec((NBLK, D), lambda i: (i, 0))],
        out_specs=[pl.BlockSpec((NBLK, D), lambda i: (i, 0)),
                   pl.BlockSpec((NBLK, D), lambda i: (i, 0))],
        out_shape=[jax.ShapeDtypeStruct((NPAD, D), jnp.float32),
                   jax.ShapeDtypeStruct((NPAD, D), jnp.float32)],
    )(y, dinv, s)


def _final_body(y_ref, dinv_ref, s_ref, o_ref):
    o_ref[...] = (s_ref[...] + y_ref[...] * dinv_ref[...]) * 0.25


def _final_mean(y, dinv, s):
    return pl.pallas_call(
        _final_body,
        grid=(NB,),
        in_specs=[pl.BlockSpec((NBLK, D), lambda i: (i, 0)),
                  pl.BlockSpec((NBLK, 1), lambda i: (i, 0)),
                  pl.BlockSpec((NBLK, D), lambda i: (i, 0))],
        out_specs=pl.BlockSpec((NBLK, D), lambda i: (i, 0)),
        out_shape=jax.ShapeDtypeStruct((NPAD, D), jnp.float32),
    )(y, dinv, s)


@functools.partial(
    pl.kernel,
    out_type=jax.ShapeDtypeStruct((32, NPAD // 128, 128), jnp.float32),
    mesh=_mesh,
    scratch_types=[pltpu.VMEM((NPAD // 128, 128), jnp.float32),
                   pltpu.VMEM((8, 128), jnp.int32)],
    compiler_params=pltpu.CompilerParams(needs_layout_passes=False,
                                         use_tc_tiling_on_sc=False),
)
def _deg_kernel(dst_hbm, degp_hbm, degpart, dstbuf):
    cid = lax.axis_index("c")
    sid = lax.axis_index("s")
    w = cid * 16 + sid

    z16 = jnp.zeros((16,), jnp.float32)

    def zero_body(i, _):
        for k in range(8):
            degpart[i, pl.ds(k * 16, 16)] = z16
        return 0

    lax.fori_loop(0, NPAD // 128, zero_body, 0)

    ones = jnp.ones((16,), jnp.float32)

    def chunk_body(n, _):
        row0 = w * 200 + n * 8
        pltpu.sync_copy(dst_hbm.at[pl.ds(row0, 8)], dstbuf)
        for r in range(8):
            for k in range(8):
                d = dstbuf[r, pl.ds(k * 16, 16)]
                plsc.addupdate_scatter(degpart, [d >> 7, d & 127], ones)
        return 0

    lax.fori_loop(0, 25, chunk_body, 0)
    pltpu.sync_copy(degpart, degp_hbm.at[w])


NCH = 13
CROWS = NCH * 8


@functools.partial(
    pl.kernel,
    out_type=(jax.ShapeDtypeStruct((2, 32, CROWS, 128), jnp.int32),
              jax.ShapeDtypeStruct((2, 32, CROWS, 128), jnp.int32),
              jax.ShapeDtypeStruct((2, 32, 16), jnp.int32)),
    mesh=_mesh,
    scratch_types=[pltpu.VMEM((8, 128), jnp.int32),
                   pltpu.VMEM((8, 128), jnp.int32),
                   pltpu.VMEM((CROWS, 128), jnp.int32),
                   pltpu.VMEM((CROWS, 128), jnp.int32),
                   pltpu.VMEM((CROWS, 128), jnp.int32),
                   pltpu.VMEM((CROWS, 128), jnp.int32),
                   pltpu.VMEM((16,), jnp.int32)],
    compiler_params=pltpu.CompilerParams(needs_layout_passes=False,
                                         use_tc_tiling_on_sc=False),
)
def _bin_kernel(src_hbm, dst_hbm, bsrc_hbm, bdl_hbm, cnt_hbm,
                sbuf, dbuf, s0, d0, s1, d1, cntbuf):
    cid = lax.axis_index("c")
    sid = lax.axis_index("s")
    w = cid * 16 + sid

    zfill = jnp.zeros((16,), jnp.int32)
    tfill = jnp.full((16,), TRASH, jnp.int32)

    def fill_body(i, _):
        for k in range(8):
            s0[i, pl.ds(k * 16, 16)] = zfill
            s1[i, pl.ds(k * 16, 16)] = zfill
            d0[i, pl.ds(k * 16, 16)] = tfill
            d1[i, pl.ds(k * 16, 16)] = tfill
        return 0

    lax.fori_loop(0, CROWS, fill_body, 0)

    halfv = jnp.full((16,), HALF, jnp.int32)
    nv = jnp.full((16,), N, jnp.int32)
    c127 = jnp.full((16,), 127, jnp.int32)
    maxpos = jnp.full((16,), NCH * 1024 - 1, jnp.int32)

    def chunk_body(n, offs):
        off0, off1 = offs
        row0 = w * 200 + n * 8
        pltpu.sync_copy(src_hbm.at[pl.ds(row0, 8)], sbuf)
        pltpu.sync_copy(dst_hbm.at[pl.ds(row0, 8)], dbuf)
        for r in range(8):
            for k in range(8):
                s = sbuf[r, pl.ds(k * 16, 16)]
                d = dbuf[r, pl.ds(k * 16, 16)]
                m0 = d < halfv
                m1 = jnp.logical_and(d >= halfv, d < nv)

                c0 = plsc.cumsum(m0.astype(jnp.int32))
                pos = jnp.minimum(c0 + (off0 - 1), maxpos)
                prow = pos >> 7
                pcol = pos & c127
                plsc.store_scatter(s0, [prow, pcol], s, mask=m0)
                plsc.store_scatter(d0, [prow, pcol], d, mask=m0)
                off0 = off0 + jnp.max(c0)

                c1 = plsc.cumsum(m1.astype(jnp.int32))
                pos = jnp.minimum(c1 + (off1 - 1), maxpos)
                prow = pos >> 7
                pcol = pos & c127
                plsc.store_scatter(s1, [prow, pcol], s, mask=m1)
                plsc.store_scatter(d1, [prow, pcol], d - halfv, mask=m1)
                off1 = off1 + jnp.max(c1)
        return (off0, off1)

    z = jnp.int32(0)
    off0, off1 = lax.fori_loop(0, 25, chunk_body, (z, z))

    pltpu.sync_copy(s0, bsrc_hbm.at[0, w])
    pltpu.sync_copy(d0, bdl_hbm.at[0, w])
    pltpu.sync_copy(s1, bsrc_hbm.at[1, w])
    pltpu.sync_copy(d1, bdl_hbm.at[1, w])
    cntbuf[pl.ds(0, 16)] = jnp.zeros((16,), jnp.int32) + ((off0 + 1023) >> 10)
    pltpu.sync_copy(cntbuf, cnt_hbm.at[0, w])
    cntbuf[pl.ds(0, 16)] = jnp.zeros((16,), jnp.int32) + ((off1 + 1023) >> 10)
    pltpu.sync_copy(cntbuf, cnt_hbm.at[1, w])


@functools.partial(
    pl.kernel,
    out_type=jax.ShapeDtypeStruct((NPAD, D), jnp.float32),
    mesh=_mesh,
    scratch_types=[pltpu.VMEM_SHARED((ACC_ROWS, D), jnp.float32),
                   pltpu.VMEM((8, 128), jnp.int32),
                   pltpu.VMEM((8, 128), jnp.int32),
                   pltpu.VMEM((2, 128, D), jnp.float32),
                   pltpu.VMEM((16,), jnp.int32),
                   pltpu.SemaphoreType.DMA,
                   pltpu.SemaphoreType.DMA],
    compiler_params=pltpu.CompilerParams(needs_layout_passes=False,
                                         use_tc_tiling_on_sc=False),
)
def _layer_kernel(zeros_hbm, z_hbm, bsrc_hbm, bdl_hbm, cnt_hbm, y_hbm,
                  acc, srcbuf, dlbuf, rowbuf, cntbuf, gsem, ssem):
    cid = lax.axis_index("c")
    sid = lax.axis_index("s")

    pltpu.sync_copy(zeros_hbm, acc.at[pl.ds(sid * 1564, 1564)])
    plsc.subcore_barrier()

    for woff in (0, 16):
        w = sid + woff
        pltpu.sync_copy(cnt_hbm.at[cid, w], cntbuf)
        nch = jnp.max(cntbuf[...])

        def chunk_body(n, _):
            @pl.when(n < nch)
            def _process():
                pltpu.sync_copy(bsrc_hbm.at[cid, w, pl.ds(n * 8, 8)], srcbuf)
                pltpu.sync_copy(bdl_hbm.at[cid, w, pl.ds(n * 8, 8)], dlbuf)
                for m in range(4):
                    gd = [pltpu.async_copy(z_hbm.at[srcbuf.at[2 * m + r]],
                                           rowbuf.at[r], gsem)
                          for r in range(2)]
                    for dsc in gd:
                        dsc.wait()
                    sd = [pltpu.async_copy(rowbuf.at[r],
                                           acc.at[dlbuf.at[2 * m + r]], ssem,
                                           add=True)
                          for r in range(2)]
                    for dsc in sd:
                        dsc.wait()
            return 0

        lax.fori_loop(0, NCH, chunk_body, 0)

    plsc.subcore_barrier()

    lo = jnp.minimum(sid * 1568, HALF - 1568)
    pltpu.sync_copy(acc.at[pl.ds(lo, 1568)],
                    y_hbm.at[pl.ds(cid * HALF + lo, 1568)])


def kernel(edge_index, user_emb, book_bert, W):
    src = edge_index[0].astype(jnp.int32)
    dst = edge_index[1].astype(jnp.int32)
    npadE = EPAD - E
    src2d = jnp.concatenate(
        [src, jnp.zeros((npadE,), jnp.int32)]).reshape(ROWS, 128)
    dst2d = jnp.concatenate(
        [dst, jnp.full((npadE,), N, jnp.int32)]).reshape(ROWS, 128)

    book_emb = _book_proj(book_bert, W.T)
    emb0 = jnp.concatenate([user_emb, book_emb], axis=0)
    emb0p = jnp.pad(emb0, ((0, NPAD - N), (0, 0)))

    degp = _deg_kernel(dst2d)
    bsrc, bdl, cnt = _bin_kernel(src2d, dst2d)
    degp_t = jnp.reshape(degp, (32, NPAD)).T
    dinv, z = _dinv_z0(degp_t, emb0p)

    s = emb0p
    zeros = jnp.zeros((1564, D), jnp.float32)
    for layer in range(NUM_LAYERS):
        y = _layer_kernel(zeros, z, bsrc, bdl, cnt)
        if layer < NUM_LAYERS - 1:
            s, z = _scale_accum(y, dinv, s)
        else:
            out = _final_mean(y, dinv, s)
    return (emb0, out[:N])


# speedup vs baseline: 1.3906x; 1.0599x over previous
import functools

import jax
import jax.numpy as jnp
from jax import lax
from jax.experimental import pallas as pl
from jax.experimental.pallas import tpu as pltpu
from jax.experimental.pallas import tpu_sc as plsc

NUM_USERS = 30000
NUM_ITEMS = 20000
D = 64
SBERT = 384
NUM_LAYERS = 3
E = 800000
N = NUM_USERS + NUM_ITEMS
NPAD = 51200
HALF = 25000
TRASH = 25000
ACC_ROWS = 25024
ROWS = 6400
EPAD = ROWS * 128
NB = 8
NBLK = NPAD // NB

_mesh = plsc.VectorSubcoreMesh(core_axis_name="c", subcore_axis_name="s",
                               num_cores=2, num_subcores=16)


def _proj_body(x_ref, w_ref, o_ref):
    o_ref[...] = jnp.dot(x_ref[...], w_ref[...],
                         preferred_element_type=jnp.float32,
                         precision=lax.Precision.HIGHEST)


def _book_proj(book_bert, w_t):
    return pl.pallas_call(
        _proj_body,
        grid=(10,),
        in_specs=[pl.BlockSpec((2000, SBERT), lambda i: (i, 0)),
                  pl.BlockSpec((SBERT, D), lambda i: (0, 0))],
        out_specs=pl.BlockSpec((2000, D), lambda i: (i, 0)),
        out_shape=jax.ShapeDtypeStruct((NUM_ITEMS, D), jnp.float32),
    )(book_bert, w_t)


def _dinv_body(degp_ref, emb_ref, dinv_ref, z_ref):
    deg = jnp.sum(degp_ref[...], axis=1, keepdims=True)
    dinv = jnp.where(deg > 0, lax.rsqrt(deg), 0.0)
    dinv_ref[...] = dinv
    z_ref[...] = emb_ref[...] * dinv


def _dinv_z0(degp, emb0p):
    return pl.pallas_call(
        _dinv_body,
        grid=(NB,),
        in_specs=[pl.BlockSpec((NBLK, 32), lambda i: (i, 0)),
                  pl.BlockSpec((NBLK, D), lambda i: (i, 0))],
        out_specs=[pl.BlockSpec((NBLK, 1), lambda i: (i, 0)),
                   pl.BlockSpec((NBLK, D), lambda i: (i, 0))],
        out_shape=[jax.ShapeDtypeStruct((NPAD, 1), jnp.float32),
                   jax.ShapeDtypeStruct((NPAD, D), jnp.float32)],
    )(degp, emb0p)


def _scale_body(y_ref, dinv_ref, s_ref, so_ref, z_ref):
    x = y_ref[...] * dinv_ref[...]
    so_ref[...] = s_ref[...] + x
    z_ref[...] = x * dinv_ref[...]


def _scale_accum(y, dinv, s):
    return pl.pallas_call(
        _scale_body,
        grid=(NB,),
        in_specs=[pl.BlockSpec((NBLK, D), lambda i: (i, 0)),
                  pl.BlockSpec((NBLK, 1), lambda i: (i, 0)),
                  pl.BlockSpec((NBLK, D), lambda i: (i, 0))],
        out_specs=[pl.BlockSpec((NBLK, D), lambda i: (i, 0)),
                   pl.BlockSpec((NBLK, D), lambda i: (i, 0))],
        out_shape=[jax.ShapeDtypeStruct((NPAD, D), jnp.float32),
                   jax.ShapeDtypeStruct((NPAD, D), jnp.float32)],
    )(y, dinv, s)


def _final_body(y_ref, dinv_ref, s_ref, o_ref):
    o_ref[...] = (s_ref[...] + y_ref[...] * dinv_ref[...]) * 0.25


def _final_mean(y, dinv, s):
    return pl.pallas_call(
        _final_body,
        grid=(NB,),
        in_specs=[pl.BlockSpec((NBLK, D), lambda i: (i, 0)),
                  pl.BlockSpec((NBLK, 1), lambda i: (i, 0)),
                  pl.BlockSpec((NBLK, D), lambda i: (i, 0))],
        out_specs=pl.BlockSpec((NBLK, D), lambda i: (i, 0)),
        out_shape=jax.ShapeDtypeStruct((NPAD, D), jnp.float32),
    )(y, dinv, s)


@functools.partial(
    pl.kernel,
    out_type=jax.ShapeDtypeStruct((32, NPAD // 128, 128), jnp.float32),
    mesh=_mesh,
    scratch_types=[pltpu.VMEM((NPAD // 128, 128), jnp.float32),
                   pltpu.VMEM((8, 128), jnp.int32)],
    compiler_params=pltpu.CompilerParams(needs_layout_passes=False,
                                         use_tc_tiling_on_sc=False),
)
def _deg_kernel(dst_hbm, degp_hbm, degpart, dstbuf):
    cid = lax.axis_index("c")
    sid = lax.axis_index("s")
    w = cid * 16 + sid

    z16 = jnp.zeros((16,), jnp.float32)

    def zero_body(i, _):
        for k in range(8):
            degpart[i, pl.ds(k * 16, 16)] = z16
        return 0

    lax.fori_loop(0, NPAD // 128, zero_body, 0)

    ones = jnp.ones((16,), jnp.float32)

    def chunk_body(n, _):
        row0 = w * 200 + n * 8
        pltpu.sync_copy(dst_hbm.at[pl.ds(row0, 8)], dstbuf)
        for r in range(8):
            for k in range(8):
                d = dstbuf[r, pl.ds(k * 16, 16)]
                plsc.addupdate_scatter(degpart, [d >> 7, d & 127], ones)
        return 0

    lax.fori_loop(0, 25, chunk_body, 0)
    pltpu.sync_copy(degpart, degp_hbm.at[w])


NCH = 13
CROWS = NCH * 8


@functools.partial(
    pl.kernel,
    out_type=(jax.ShapeDtypeStruct((2, 32, CROWS, 128), jnp.int32),
              jax.ShapeDtypeStruct((2, 32, CROWS, 128), jnp.int32),
              jax.ShapeDtypeStruct((2, 32, 16), jnp.int32)),
    mesh=_mesh,
    scratch_types=[pltpu.VMEM((8, 128), jnp.int32),
                   pltpu.VMEM((8, 128), jnp.int32),
                   pltpu.VMEM((CROWS, 128), jnp.int32),
                   pltpu.VMEM((CROWS, 128), jnp.int32),
                   pltpu.VMEM((CROWS, 128), jnp.int32),
                   pltpu.VMEM((CROWS, 128), jnp.int32),
                   pltpu.VMEM((16,), jnp.int32)],
    compiler_params=pltpu.CompilerParams(needs_layout_passes=False,
                                         use_tc_tiling_on_sc=False),
)
def _bin_kernel(src_hbm, dst_hbm, bsrc_hbm, bdl_hbm, cnt_hbm,
                sbuf, dbuf, s0, d0, s1, d1, cntbuf):
    cid = lax.axis_index("c")
    sid = lax.axis_index("s")
    w = cid * 16 + sid

    zfill = jnp.zeros((16,), jnp.int32)
    tfill = jnp.full((16,), TRASH, jnp.int32)

    def fill_body(i, _):
        for k in range(8):
            s0[i, pl.ds(k * 16, 16)] = zfill
            s1[i, pl.ds(k * 16, 16)] = zfill
            d0[i, pl.ds(k * 16, 16)] = tfill
            d1[i, pl.ds(k * 16, 16)] = tfill
        return 0

    lax.fori_loop(0, CROWS, fill_body, 0)

    halfv = jnp.full((16,), HALF, jnp.int32)
    nv = jnp.full((16,), N, jnp.int32)
    c127 = jnp.full((16,), 127, jnp.int32)
    maxpos = jnp.full((16,), NCH * 1024 - 1, jnp.int32)

    def chunk_body(n, offs):
        off0, off1 = offs
        row0 = w * 200 + n * 8
        pltpu.sync_copy(src_hbm.at[pl.ds(row0, 8)], sbuf)
        pltpu.sync_copy(dst_hbm.at[pl.ds(row0, 8)], dbuf)
        for r in range(8):
            for k in range(8):
                s = sbuf[r, pl.ds(k * 16, 16)]
                d = dbuf[r, pl.ds(k * 16, 16)]
                m0 = d < halfv
                m1 = jnp.logical_and(d >= halfv, d < nv)

                c0 = plsc.cumsum(m0.astype(jnp.int32))
                pos = jnp.minimum(c0 + (off0 - 1), maxpos)
                prow = pos >> 7
                pcol = pos & c127
                plsc.store_scatter(s0, [prow, pcol], s, mask=m0)
                plsc.store_scatter(d0, [prow, pcol], d, mask=m0)
                off0 = off0 + jnp.max(c0)

                c1 = plsc.cumsum(m1.astype(jnp.int32))
                pos = jnp.minimum(c1 + (off1 - 1), maxpos)
                prow = pos >> 7
                pcol = pos & c127
                plsc.store_scatter(s1, [prow, pcol], s, mask=m1)
                plsc.store_scatter(d1, [prow, pcol], d - halfv, mask=m1)
                off1 = off1 + jnp.max(c1)
        return (off0, off1)

    z = jnp.int32(0)
    off0, off1 = lax.fori_loop(0, 25, chunk_body, (z, z))

    pltpu.sync_copy(s0, bsrc_hbm.at[0, w])
    pltpu.sync_copy(d0, bdl_hbm.at[0, w])
    pltpu.sync_copy(s1, bsrc_hbm.at[1, w])
    pltpu.sync_copy(d1, bdl_hbm.at[1, w])
    cntbuf[pl.ds(0, 16)] = jnp.zeros((16,), jnp.int32) + ((off0 + 1023) >> 10)
    pltpu.sync_copy(cntbuf, cnt_hbm.at[0, w])
    cntbuf[pl.ds(0, 16)] = jnp.zeros((16,), jnp.int32) + ((off1 + 1023) >> 10)
    pltpu.sync_copy(cntbuf, cnt_hbm.at[1, w])


@functools.partial(
    pl.kernel,
    out_type=jax.ShapeDtypeStruct((NPAD, D), jnp.float32),
    mesh=_mesh,
    scratch_types=[pltpu.VMEM_SHARED((ACC_ROWS, D), jnp.float32),
                   pltpu.VMEM((8, 128), jnp.int32),
                   pltpu.VMEM((8, 128), jnp.int32),
                   pltpu.VMEM((3, 128, D), jnp.float32),
                   pltpu.VMEM((16,), jnp.int32),
                   pltpu.SemaphoreType.DMA,
                   pltpu.SemaphoreType.DMA],
    compiler_params=pltpu.CompilerParams(needs_layout_passes=False,
                                         use_tc_tiling_on_sc=False),
)
def _layer_kernel(zeros_hbm, z_hbm, bsrc_hbm, bdl_hbm, cnt_hbm, y_hbm,
                  acc, srcbuf, dlbuf, rowbuf, cntbuf, gsem, ssem):
    cid = lax.axis_index("c")
    sid = lax.axis_index("s")

    pltpu.sync_copy(zeros_hbm, acc.at[pl.ds(sid * 1564, 1564)])
    plsc.subcore_barrier()

    for woff in (0, 16):
        w = sid + woff
        pltpu.sync_copy(cnt_hbm.at[cid, w], cntbuf)
        nch = jnp.max(cntbuf[...])

        def chunk_body(n, _):
            @pl.when(n < nch)
            def _process():
                pltpu.sync_copy(bsrc_hbm.at[cid, w, pl.ds(n * 8, 8)], srcbuf)
                pltpu.sync_copy(bdl_hbm.at[cid, w, pl.ds(n * 8, 8)], dlbuf)
                g = [None] * 8
                s = [None] * 8
                for r in range(2):
                    g[r] = pltpu.async_copy(z_hbm.at[srcbuf.at[r]],
                                            rowbuf.at[r], gsem)
                for r in range(8):
                    g[r].wait()
                    if r >= 1:
                        s[r - 1].wait()
                    if r + 2 < 8:
                        g[r + 2] = pltpu.async_copy(
                            z_hbm.at[srcbuf.at[r + 2]],
                            rowbuf.at[(r + 2) % 3], gsem)
                    s[r] = pltpu.async_copy(rowbuf.at[r % 3],
                                            acc.at[dlbuf.at[r]], ssem,
                                            add=True)
                s[7].wait()
            return 0

        lax.fori_loop(0, NCH, chunk_body, 0)

    plsc.subcore_barrier()

    lo = jnp.minimum(sid * 1568, HALF - 1568)
    pltpu.sync_copy(acc.at[pl.ds(lo, 1568)],
                    y_hbm.at[pl.ds(cid * HALF + lo, 1568)])


def kernel(edge_index, user_emb, book_bert, W):
    src = edge_index[0].astype(jnp.int32)
    dst = edge_index[1].astype(jnp.int32)
    npadE = EPAD - E
    src2d = jnp.concatenate(
        [src, jnp.zeros((npadE,), jnp.int32)]).reshape(ROWS, 128)
    dst2d = jnp.concatenate(
        [dst, jnp.full((npadE,), N, jnp.int32)]).reshape(ROWS, 128)

    book_emb = _book_proj(book_bert, W.T)
    emb0 = jnp.concatenate([user_emb, book_emb], axis=0)
    emb0p = jnp.pad(emb0, ((0, NPAD - N), (0, 0)))

    degp = _deg_kernel(dst2d)
    bsrc, bdl, cnt = _bin_kernel(src2d, dst2d)
    degp_t = jnp.reshape(degp, (32, NPAD)).T
    dinv, z = _dinv_z0(degp_t, emb0p)

    s = emb0p
    zeros = jnp.zeros((1564, D), jnp.float32)
    for layer in range(NUM_LAYERS):
        y = _layer_kernel(zeros, z, bsrc, bdl, cnt)
        if layer < NUM_LAYERS - 1:
            s, z = _scale_accum(y, dinv, s)
        else:
            out = _final_mean(y, dinv, s)
    return (emb0, out[:N])
